# Initial kernel scaffold; baseline (speedup 1.0000x reference)
import jax
import jax.numpy as jnp
from jax.experimental import pallas as pl


def kernel(x, pos_table):
    raise NotImplementedError("write your pallas kernel here")


# speedup vs baseline: 1.9357x; 1.9357x over previous
import jax
import jax.numpy as jnp
from jax.experimental import pallas as pl


_BS = 256


def _body(x_ref, t_ref, o_ref):
    o_ref[...] = x_ref[...] + t_ref[...][None, :, :]


def kernel(x, pos_table):
    B, S, D = x.shape
    bs = _BS if S % _BS == 0 else S
    return pl.pallas_call(
        _body,
        grid=(S // bs,),
        in_specs=[
            pl.BlockSpec((B, bs, D), lambda i: (0, i, 0)),
            pl.BlockSpec((bs, D), lambda i: (i, 0)),
        ],
        out_specs=pl.BlockSpec((B, bs, D), lambda i: (0, i, 0)),
        out_shape=jax.ShapeDtypeStruct((B, S, D), x.dtype),
    )(x, pos_table)


# speedup vs baseline: 1.9636x; 1.0144x over previous
import jax
import jax.numpy as jnp
from jax.experimental import pallas as pl


_BS = 512


def _body(x_ref, t_ref, o_ref):
    o_ref[...] = x_ref[...] + t_ref[...][None, :, :]


def kernel(x, pos_table):
    B, S, D = x.shape
    bs = _BS if S % _BS == 0 else S
    return pl.pallas_call(
        _body,
        grid=(S // bs,),
        in_specs=[
            pl.BlockSpec((B, bs, D), lambda i: (0, i, 0)),
            pl.BlockSpec((bs, D), lambda i: (i, 0)),
        ],
        out_specs=pl.BlockSpec((B, bs, D), lambda i: (0, i, 0)),
        out_shape=jax.ShapeDtypeStruct((B, S, D), x.dtype),
    )(x, pos_table)
